# baseline (speedup 1.0000x reference)
import functools

import jax
import jax.numpy as jnp
from jax import lax
from jax.experimental import pallas as pl
from jax.experimental.pallas import tpu as pltpu
from jax.experimental.pallas import tpu_sc as plsc

_TOTAL = 8 * 3136 * 768
_NC, _NS = 2, 16
_NW = _NC * _NS
_WCH = _TOTAL // _NW
_CH = 50176
_NCH = _WCH // _CH


def _sc_copy_body(x_hbm, o_hbm, b0, b1, si0, si1, so0, so1):
    wid = lax.axis_index("s") * _NC + lax.axis_index("c")
    base = wid * _WCH
    bufs, isems, osems = (b0, b1), (si0, si1), (so0, so1)

    def in_cp(i, b):
        return pltpu.make_async_copy(
            x_hbm.at[pl.ds(base + i * _CH, _CH)], bufs[b], isems[b])

    def out_cp(i, b):
        return pltpu.make_async_copy(
            bufs[b], o_hbm.at[pl.ds(base + i * _CH, _CH)], osems[b])

    in_cp(0, 0).start()
    for i in range(_NCH):
        b = i % 2
        if i + 1 < _NCH:
            if i >= 1:
                out_cp(i - 1, 1 - b).wait()
            in_cp(i + 1, 1 - b).start()
        in_cp(i, b).wait()
        out_cp(i, b).start()
    out_cp(_NCH - 2, _NCH % 2).wait()
    out_cp(_NCH - 1, 1 - _NCH % 2).wait()


_sc_copy = functools.partial(
    pl.kernel,
    out_type=jax.ShapeDtypeStruct((_TOTAL,), jnp.float32),
    mesh=plsc.VectorSubcoreMesh(core_axis_name="c", subcore_axis_name="s"),
    scratch_types=[
        pltpu.VMEM((_CH,), jnp.float32),
        pltpu.VMEM((_CH,), jnp.float32),
        pltpu.SemaphoreType.DMA,
        pltpu.SemaphoreType.DMA,
        pltpu.SemaphoreType.DMA,
        pltpu.SemaphoreType.DMA,
    ],
)(_sc_copy_body)


def kernel(x):
    flat = x.reshape(_TOTAL)
    return _sc_copy(flat).reshape(x.shape)


# speedup vs baseline: 4.7830x; 4.7830x over previous
import jax
import jax.numpy as jnp
from jax.experimental import pallas as pl
from jax.experimental.pallas import tpu as pltpu

_ROWS = 8 * 3136
_COLS = 768
_CH = 4480
_NCH = -(-_ROWS // _CH)
_CHUNKS = [(i * _CH, min(_CH, _ROWS - i * _CH)) for i in range(_NCH)]
_NBUF = 4


def _ring_body(x_hbm, o_hbm, b0, b1, b2, b3, i0, i1, i2, i3, o0, o1, o2, o3):
    bufs = (b0, b1, b2, b3)
    ise = (i0, i1, i2, i3)
    ose = (o0, o1, o2, o3)

    def cin(i):
        off, n = _CHUNKS[i]
        b = i % _NBUF
        return pltpu.make_async_copy(
            x_hbm.at[pl.ds(off, n)], bufs[b].at[pl.ds(0, n)], ise[b])

    def cout(i):
        off, n = _CHUNKS[i]
        b = i % _NBUF
        return pltpu.make_async_copy(
            bufs[b].at[pl.ds(0, n)], o_hbm.at[pl.ds(off, n)], ose[b])

    for i in range(_NBUF):
        cin(i).start()
    for i in range(_NCH):
        cin(i).wait()
        cout(i).start()
        if i >= 1 and i + _NBUF - 1 < _NCH:
            cout(i - 1).wait()
            cin(i + _NBUF - 1).start()
    for i in range(2, _NCH):
        cout(i).wait()


def kernel(x):
    flat = x.reshape(_ROWS, _COLS)
    out = pl.pallas_call(
        _ring_body,
        out_shape=jax.ShapeDtypeStruct(flat.shape, flat.dtype),
        in_specs=[pl.BlockSpec(memory_space=pl.ANY)],
        out_specs=pl.BlockSpec(memory_space=pl.ANY),
        scratch_shapes=(
            [pltpu.VMEM((_CH, _COLS), jnp.float32) for _ in range(_NBUF)]
            + [pltpu.SemaphoreType.DMA] * (2 * _NBUF)
        ),
        compiler_params=pltpu.CompilerParams(vmem_limit_bytes=128 * 1024 * 1024),
    )(flat)
    return out.reshape(x.shape)
